# baseline (speedup 1.0000x reference)
import jax
import jax.numpy as jnp
from jax.experimental import pallas as pl

DECAY = 0.95
DENSITY_SCALE = 1.0


def kernel(density_grid, sigmas, indices):
    C, Mc = density_grid.shape
    sig = (sigmas * DENSITY_SCALE).reshape(-1)
    offsets = (jnp.arange(C, dtype=indices.dtype) * Mc)[:, None]
    flat_idx = (indices + offsets).reshape(-1)
    pos = jnp.arange(flat_idx.shape[0], dtype=jnp.int32)
    win = jnp.full((C * Mc,), -1, dtype=jnp.int32).at[flat_idx].max(pos)
    hit = win >= 0
    tmp = jnp.where(hit, sig[jnp.maximum(win, 0)], -1.0).reshape(C, Mc)
    valid = (density_grid >= 0) & (tmp >= 0)
    return jnp.where(valid, jnp.maximum(density_grid * DECAY, tmp), density_grid)


# speedup vs baseline: 49.1584x; 49.1584x over previous
import functools

import jax
import jax.numpy as jnp
from jax import lax
from jax.experimental import pallas as pl
from jax.experimental.pallas import tpu as pltpu
from jax.experimental.pallas import tpu_sc as plsc

CASCADE = 4
GRID = 128
M = GRID ** 3
N = M // 4 * 2
DECAY = 0.95
DENSITY_SCALE = 1.0

NC = 2
NS = 16
NW = NC * NS
SLICE = M // NW
WSZ = 4096
NWIN = N // WSZ


def _sc_body(idx_hbm, sig_hbm, tmp_hbm, tmp_v, ibuf, sbuf):
    core = lax.axis_index("c")
    sub = lax.axis_index("s")
    wid = sub * NC + core
    base = wid * SLICE

    for cc in range(CASCADE):
        c = lax.rem(wid + cc, CASCADE)
        elem0 = c * N

        def _ms(i, carry):
            for u in range(8):
                tmp_v[pl.ds((i * 8 + u) * 16, 16)] = jnp.full(
                    (16,), -1.0, jnp.float32)
            return carry
        lax.fori_loop(0, SLICE // (16 * 8), _ms, 0)

        def _win(w, carry):
            off = elem0 + w * WSZ
            pltpu.sync_copy(idx_hbm.at[pl.ds(off, WSZ)], ibuf)
            pltpu.sync_copy(sig_hbm.at[pl.ds(off, WSZ)], sbuf)

            def _vec(v, c2):
                iv = ibuf[pl.ds(v * 16, 16)]
                sv = sbuf[pl.ds(v * 16, 16)]
                loc = iv - base
                ok = (loc >= 0) & (loc < SLICE)
                locc = jnp.where(ok, loc, 0)
                _, last = plsc.scan_count(locc, mask=ok)
                plsc.store_scatter(tmp_v, [locc], sv, mask=last)
                return c2
            lax.fori_loop(0, WSZ // 16, _vec, 0)
            return carry
        lax.fori_loop(0, NWIN, _win, 0)

        pltpu.sync_copy(tmp_v, tmp_hbm.at[pl.ds(c * M + base, SLICE)])


@jax.jit
def _sc_scatter(idx_flat, sig_flat):
    kfn = functools.partial(
        pl.kernel,
        out_type=jax.ShapeDtypeStruct((CASCADE * M,), jnp.float32),
        mesh=plsc.VectorSubcoreMesh(core_axis_name="c", subcore_axis_name="s"),
        compiler_params=pltpu.CompilerParams(needs_layout_passes=False),
        scratch_types=[
            pltpu.VMEM((SLICE,), jnp.float32),
            pltpu.VMEM((WSZ,), jnp.int32),
            pltpu.VMEM((WSZ,), jnp.float32),
        ],
    )(_sc_body)
    return kfn(idx_flat, sig_flat)


def _tc_body(g_ref, t_ref, o_ref):
    g = g_ref[...]
    t = t_ref[...]
    o_ref[...] = jnp.where((g >= 0.0) & (t >= 0.0),
                           jnp.maximum(g * DECAY, t), g)


@jax.jit
def _tc_combine(density_grid, tmp_grid):
    C, Mc = density_grid.shape
    rows, cols = 8192, C * Mc // 8192
    blk = 512
    out = pl.pallas_call(
        _tc_body,
        out_shape=jax.ShapeDtypeStruct((rows, cols), jnp.float32),
        grid=(rows // blk,),
        in_specs=[
            pl.BlockSpec((blk, cols), lambda i: (i, 0)),
            pl.BlockSpec((blk, cols), lambda i: (i, 0)),
        ],
        out_specs=pl.BlockSpec((blk, cols), lambda i: (i, 0)),
    )(density_grid.reshape(rows, cols), tmp_grid.reshape(rows, cols))
    return out.reshape(C, Mc)


def kernel(density_grid, sigmas, indices):
    C, Mc = density_grid.shape
    idx_flat = indices.reshape(-1)
    sig_flat = (sigmas * DENSITY_SCALE).reshape(-1)
    tmp = _sc_scatter(idx_flat, sig_flat).reshape(C, Mc)
    return _tc_combine(density_grid, tmp)
